# baseline (speedup 1.0000x reference)
import functools

import jax
import jax.numpy as jnp
from jax import lax
from jax.experimental import pallas as pl
from jax.experimental.pallas import tpu as pltpu
from jax.experimental.pallas import tpu_sc as plsc

f32 = jnp.float32

D = 128
NC, NS, L = 2, 16, 16
NW = NC * NS
CHUNK = 128
UNROLL = 4
NPAD = 10240
SLAB = NPAD // NS
N = 10000
TRASH = N


def _sc_body(with_cnt, n_chunks, ept, *refs):
  if with_cnt:
    (h, srci, dsti, zrows, orows,
     out, cntout,
     s0, s1, s2, s3, d0, d1, d2, d3, rows0, rows1, acc,
     i0, i1, i2, i3, g0, g1, ss0, ss1) = refs
  else:
    (h, srci, dsti, zrows,
     out,
     s0, s1, s2, s3, d0, d1, d2, d3, rows0, rows1, acc,
     i0, i1, i2, i3, g0, g1, ss0, ss1) = refs
    cntout = orows = None
  sidx = (s0, s1, s2, s3)
  didx = (d0, d1, d2, d3)
  rows = (rows0, rows1)
  isem = (i0, i1, i2, i3)
  gsem = (g0, g1)
  ssem = (ss0, ss1)
  c = lax.axis_index("c")
  s = lax.axis_index("s")
  t = c * NS + s
  n = n_chunks

  def ibase(i):
    return t * ept + i * CHUNK

  def iload(i, q, src_too=True):
    if src_too:
      pltpu.async_copy(srci.at[pl.ds(ibase(i), CHUNK)], sidx[q], isem[q])
    pltpu.async_copy(dsti.at[pl.ds(ibase(i), CHUNK)], didx[q], isem[q])

  def iwait(i, q, src_too=True):
    if src_too:
      pltpu.make_async_copy(
          srci.at[pl.ds(ibase(i), CHUNK)], sidx[q], isem[q]).wait()
    pltpu.make_async_copy(
        dsti.at[pl.ds(ibase(i), CHUNK)], didx[q], isem[q]).wait()

  def swait(b, q):
    pltpu.make_async_copy(rows[b], acc.at[didx[q]], ssem[b]).wait()

  pltpu.sync_copy(zrows, rows0)

  def zbody(j, carry):
    r = s * SLAB + j * CHUNK
    pltpu.sync_copy(rows0, acc.at[pl.ds(r, CHUNK)])
    return carry

  lax.fori_loop(0, SLAB // CHUNK, zbody, 0)
  plsc.subcore_barrier()

  iload(0, 0)
  iload(1, 1)
  iwait(0, 0)
  pltpu.async_copy(h.at[sidx[0]], rows[0], gsem[0])

  def pbody(i4, carry):
    for b4 in range(UNROLL):
      i = i4 * UNROLL + b4
      b = b4 % 2
      bn = 1 - b
      q0, q1, q2 = b4, (b4 + 1) % 4, (b4 + 2) % 4
      qp = (b4 + 3) % 4

      @pl.when(i + 1 < n)
      def _():
        iwait(i + 1, q1)

      pltpu.make_async_copy(h.at[sidx[q0]], rows[b], gsem[b]).wait()
      pltpu.async_copy(rows[b], acc.at[didx[q0]], ssem[b], add=True)

      @pl.when((i + 1 < n) & (i >= 1))
      def _():
        swait(bn, qp)

      @pl.when(i + 1 < n)
      def _():
        pltpu.async_copy(h.at[sidx[q1]], rows[bn], gsem[bn])

      @pl.when(i + 2 < n)
      def _():
        iload(i + 2, q2)
    return carry

  lax.fori_loop(0, n // UNROLL, pbody, 0)
  swait(0, (n - 2) % 4)
  swait(1, (n - 1) % 4)
  plsc.subcore_barrier()

  def obody(dst_hbm, j, carry):
    r = s * SLAB + j * CHUNK
    pltpu.sync_copy(acc.at[pl.ds(r, CHUNK)], rows0)
    pltpu.sync_copy(rows0, dst_hbm.at[c, pl.ds(r, CHUNK)])
    return carry

  lax.fori_loop(0, SLAB // CHUNK, functools.partial(obody, out), 0)

  if with_cnt:
    plsc.subcore_barrier()
    pltpu.sync_copy(zrows, rows0)
    lax.fori_loop(0, SLAB // CHUNK, zbody, 0)
    pltpu.sync_copy(orows, rows0)
    plsc.subcore_barrier()

    def cwait(b, q):
      pltpu.make_async_copy(rows0, acc.at[didx[q]], ssem[b]).wait()

    iload(0, 0, src_too=False)
    iload(1, 1, src_too=False)

    def cbody(i4, carry):
      for b4 in range(UNROLL):
        i = i4 * UNROLL + b4
        b = b4 % 2
        q0, q2 = b4, (b4 + 2) % 4
        iwait(i, q0, src_too=False)

        @pl.when(i >= 2)
        def _():
          cwait(b, q2)

        pltpu.async_copy(rows0, acc.at[didx[q0]], ssem[b], add=True)

        @pl.when(i + 2 < n)
        def _():
          iload(i + 2, q2, src_too=False)
      return carry

    lax.fori_loop(0, n // UNROLL, cbody, 0)
    cwait(0, (n - 2) % 4)
    cwait(1, (n - 1) % 4)
    plsc.subcore_barrier()
    lax.fori_loop(0, SLAB // CHUNK, functools.partial(obody, cntout), 0)


@functools.lru_cache(maxsize=None)
def _build_agg(epad, with_cnt):
  ept = epad // NW
  n_chunks = ept // CHUNK
  out_type = [jax.ShapeDtypeStruct((NC, NPAD, D), f32)]
  scratch = [pltpu.VMEM((CHUNK,), jnp.int32) for _ in range(8)]
  scratch += [pltpu.VMEM((CHUNK, D), f32), pltpu.VMEM((CHUNK, D), f32)]
  if with_cnt:
    out_type.append(jax.ShapeDtypeStruct((NC, NPAD, D), f32))
  scratch.append(pltpu.VMEM_SHARED((NPAD, D), f32))
  scratch += [pltpu.SemaphoreType.DMA for _ in range(8)]
  mesh = plsc.VectorSubcoreMesh(core_axis_name="c", subcore_axis_name="s")
  return pl.kernel(
      functools.partial(_sc_body, with_cnt, n_chunks, ept),
      out_type=out_type,
      mesh=mesh,
      scratch_types=scratch,
  )


def _tc_body(relu, p_ref, cnt_ref, h_ref, w_ref, b_ref, o_ref):
  s = p_ref[0] + p_ref[1] + h_ref[...]
  mean = s / cnt_ref[...]
  y = lax.dot_general(mean, w_ref[...], (((1,), (1,)), ((), ())),
                      preferred_element_type=f32) + b_ref[...]
  if relu:
    y = jnp.maximum(y, 0.0)
  o_ref[...] = y


BR = 2000


@functools.lru_cache(maxsize=None)
def _build_layer(relu):
  return pl.pallas_call(
      functools.partial(_tc_body, relu),
      grid=(N // BR,),
      in_specs=[
          pl.BlockSpec((NC, BR, D), lambda g: (0, g, 0)),
          pl.BlockSpec((BR, 1), lambda g: (g, 0)),
          pl.BlockSpec((BR, D), lambda g: (g, 0)),
          pl.BlockSpec((D, D), lambda g: (0, 0)),
          pl.BlockSpec((1, D), lambda g: (0, 0)),
      ],
      out_specs=pl.BlockSpec((BR, D), lambda g: (g, 0)),
      out_shape=jax.ShapeDtypeStruct((N, D), f32),
  )


def kernel(x, edge_index, W1, b1, W2, b2):
  e = edge_index.shape[1]
  step = NW * CHUNK * UNROLL
  epad = -(-e // step) * step
  src = jnp.pad(edge_index[0].astype(jnp.int32), (0, epad - e))
  dst = jnp.pad(edge_index[1].astype(jnp.int32), (0, epad - e))
  dst = jnp.where(src == dst, TRASH, dst)
  zrows = jnp.zeros((CHUNK, D), f32)
  orows = jnp.ones((CHUNK, D), f32)

  agg1, cntg = _build_agg(epad, True)(x, src, dst, zrows, orows)
  cnt = (cntg[0, :N, 0] + cntg[1, :N, 0] + 1.0).reshape(N, 1)
  h1 = _build_layer(True)(agg1, cnt, x, W1, b1.reshape(1, D))
  (agg2,) = _build_agg(epad, False)(h1, src, dst, zrows)
  return _build_layer(False)(agg2, cnt, h1, W2, b2.reshape(1, D))
